# baseline (speedup 1.0000x reference)
import jax
import jax.numpy as jnp
from jax.experimental import pallas as pl
from jax.experimental.pallas import tpu as pltpu

_ANCHOR_H = 19.15
_ANCHOR_W = 85.72
_CELL = 32.0
_NA = 3
_THETA_MARGIN = 180.0 / _NA
_H = 160
_W = 160
_HW = _H * _W
_HWBLK = 3200
_NB = _HW // _HWBLK
_CHW = 128
_NCHUNK = _HWBLK // _CHW
_CHUNK_ROWS = _NA * _CHW


def _count_body(x_ref, thr_ref, cnt_ref):
    x = x_ref[0]
    thr = thr_ref[0]
    m = None
    for a in range(_NA):
        conf = jax.nn.sigmoid(x[6 * a:6 * a + 1, :])
        ma = (conf >= thr).astype(jnp.int32)
        m = ma if m is None else m + ma
    g = m.reshape(_HW // _CHW, _CHW)
    cnt_ref[0] = jnp.sum(g, axis=1, keepdims=True)


def _cumsum_lanes(x):
    n = x.shape[1]
    lane = jax.lax.broadcasted_iota(jnp.int32, x.shape, 1)
    k = 1
    while k < n:
        x = x + jnp.where(lane >= k, pltpu.roll(x, k, axis=1), 0)
        k *= 2
    return x


def _scatter_body(offs_ref, x_ref, thr_ref, _outz_ref, out_ref,
                  dec_ref, stage_a, stage_b, sems):
    n = pl.program_id(0)
    b = pl.program_id(1)
    step = n * _NB + b
    x = x_ref[0]
    thr = thr_ref[0]

    hw_i = jax.lax.broadcasted_iota(jnp.int32, (1, _HWBLK), 1) + b * _HWBLK
    iy_i = hw_i // _W
    ix = (hw_i - iy_i * _W).astype(jnp.float32)
    iy = iy_i.astype(jnp.float32)

    masks = []
    for a in range(_NA):
        c0 = a * 6
        conf = jax.nn.sigmoid(x[c0 + 0:c0 + 1, :])
        dec_ref[a, 0:1, :] = conf
        dec_ref[a, 1:2, :] = (ix + jax.nn.sigmoid(x[c0 + 1:c0 + 2, :])) * _CELL
        dec_ref[a, 2:3, :] = (iy + jax.nn.sigmoid(x[c0 + 2:c0 + 3, :])) * _CELL
        dec_ref[a, 3:4, :] = _ANCHOR_W * jnp.exp(x[c0 + 3:c0 + 4, :])
        dec_ref[a, 4:5, :] = _ANCHOR_H * jnp.exp(x[c0 + 4:c0 + 5, :])
        dec_ref[a, 5:6, :] = (a + jax.nn.sigmoid(x[c0 + 5:c0 + 6, :])) * _THETA_MARGIN
        masks.append((conf >= thr).astype(jnp.int32).reshape(_NCHUNK, _CHW))

    m0, m1, m2 = masks
    pa1 = m0 + m1
    t = pa1 + m2
    pexc = _cumsum_lanes(t) - t
    d0 = jnp.where(m0 == 1, pexc + m0 - 1, -1)
    d1 = jnp.where(m1 == 1, pexc + pa1 - 1, -1)
    d2 = jnp.where(m2 == 1, pexc + t - 1, -1)

    iota_d = jax.lax.broadcasted_iota(jnp.int32, (_CHUNK_ROWS, _CHUNK_ROWS), 0)

    def dma(c, stage, cnt, goff):
        return pltpu.make_async_copy(
            stage.at[pl.ds(0, cnt), :],
            out_ref.at[pl.ds(goff, cnt), :],
            sems.at[c % 2],
        )

    prev = []
    for c in range(_NCHUNK):
        dest = jnp.concatenate(
            [d0[c:c + 1, :], d1[c:c + 1, :], d2[c:c + 1, :]], axis=1)
        sel = (iota_d == dest).astype(jnp.float32)
        tile6 = jnp.concatenate(
            [dec_ref[a, 0:6, c * _CHW:(c + 1) * _CHW] for a in range(_NA)],
            axis=1)
        outc = jax.lax.dot_general(
            sel, tile6, (((1,), (1,)), ((), ())),
            preferred_element_type=jnp.float32,
            precision=jax.lax.Precision.HIGHEST)
        stage = stage_a if c % 2 == 0 else stage_b
        if len(prev) >= 2:
            pc, ps, pcnt, pgoff = prev.pop(0)

            @pl.when(pcnt > 0)
            def _():
                dma(pc, ps, pcnt, pgoff).wait()
        stage[...] = outc
        i = step * _NCHUNK + c
        goff = offs_ref[i]
        cnt = offs_ref[i + 1] - goff

        @pl.when(cnt > 0)
        def _():
            dma(c, stage, cnt, goff).start()
        prev.append((c, stage, cnt, goff))

    for pc, ps, pcnt, pgoff in prev:
        @pl.when(pcnt > 0)
        def _():
            dma(pc, ps, pcnt, pgoff).wait()


def kernel(output, confidence_threshold):
    N, C, H, W = output.shape
    x = output.reshape(N, C, _HW)
    thr = confidence_threshold.reshape(1)
    nrows = N * _HW * _NA

    counts = pl.pallas_call(
        _count_body,
        grid=(N,),
        in_specs=[
            pl.BlockSpec((1, C, _HW), lambda n: (n, 0, 0)),
            pl.BlockSpec(memory_space=pltpu.SMEM),
        ],
        out_specs=pl.BlockSpec((1, _HW // _CHW, 1), lambda n: (n, 0, 0)),
        out_shape=jax.ShapeDtypeStruct((N, _HW // _CHW, 1), jnp.int32),
        compiler_params=pltpu.CompilerParams(
            dimension_semantics=("parallel",),
        ),
    )(x, thr)

    offs = jnp.concatenate(
        [jnp.zeros((1,), jnp.int32),
         jnp.cumsum(counts.reshape(-1), dtype=jnp.int32)])

    outz = jnp.zeros((nrows, 6), jnp.float32)

    compacted = pl.pallas_call(
        _scatter_body,
        grid_spec=pltpu.PrefetchScalarGridSpec(
            num_scalar_prefetch=1,
            grid=(N, _NB),
            in_specs=[
                pl.BlockSpec((1, C, _HWBLK), lambda n, b, offs: (n, 0, b)),
                pl.BlockSpec(memory_space=pltpu.SMEM),
                pl.BlockSpec(memory_space=pl.ANY),
            ],
            out_specs=pl.BlockSpec(memory_space=pl.ANY),
            scratch_shapes=[
                pltpu.VMEM((_NA, 8, _HWBLK), jnp.float32),
                pltpu.VMEM((_CHUNK_ROWS, 6), jnp.float32),
                pltpu.VMEM((_CHUNK_ROWS, 6), jnp.float32),
                pltpu.SemaphoreType.DMA((2,)),
            ],
        ),
        out_shape=jax.ShapeDtypeStruct((nrows, 6), jnp.float32),
        input_output_aliases={3: 0},
        compiler_params=pltpu.CompilerParams(
            dimension_semantics=("parallel", "arbitrary"),
        ),
    )(offs, x, thr, outz)
    return compacted


# speedup vs baseline: 1.0885x; 1.0885x over previous
import jax
import jax.numpy as jnp
from jax.experimental import pallas as pl
from jax.experimental.pallas import tpu as pltpu

_ANCHOR_H = 19.15
_ANCHOR_W = 85.72
_CELL = 32.0
_NA = 3
_THETA_MARGIN = 180.0 / _NA
_H = 160
_W = 160
_HW = _H * _W
_HWBLK = 3200
_NB = _HW // _HWBLK
_CHW = 128
_NCHUNK = _HWBLK // _CHW
_CHUNK_ROWS = _NA * _CHW
_DEPTH = 8


def _count_body(x_ref, thr_ref, cnt_ref):
    x = x_ref[0]
    thr = thr_ref[0]
    m = None
    for a in range(_NA):
        conf = jax.nn.sigmoid(x[6 * a:6 * a + 1, :])
        ma = (conf >= thr).astype(jnp.int32)
        m = ma if m is None else m + ma
    g = m.reshape(_HW // _CHW, _CHW)
    cnt_ref[0] = jnp.sum(g, axis=1, keepdims=True)


def _cumsum_lanes(x):
    n = x.shape[1]
    lane = jax.lax.broadcasted_iota(jnp.int32, x.shape, 1)
    k = 1
    while k < n:
        x = x + jnp.where(lane >= k, pltpu.roll(x, k, axis=1), 0)
        k *= 2
    return x


def _scatter_body(offs_ref, x_ref, thr_ref, _outz_ref, out_ref,
                  dec_ref, stages, sems):
    n = pl.program_id(0)
    b = pl.program_id(1)
    step = n * _NB + b
    x = x_ref[0]
    thr = thr_ref[0]

    hw_i = jax.lax.broadcasted_iota(jnp.int32, (1, _HWBLK), 1) + b * _HWBLK
    iy_i = hw_i // _W
    ix = (hw_i - iy_i * _W).astype(jnp.float32)
    iy = iy_i.astype(jnp.float32)

    masks = []
    for a in range(_NA):
        c0 = a * 6
        conf = jax.nn.sigmoid(x[c0 + 0:c0 + 1, :])
        dec_ref[a, 0:1, :] = conf
        dec_ref[a, 1:2, :] = (ix + jax.nn.sigmoid(x[c0 + 1:c0 + 2, :])) * _CELL
        dec_ref[a, 2:3, :] = (iy + jax.nn.sigmoid(x[c0 + 2:c0 + 3, :])) * _CELL
        dec_ref[a, 3:4, :] = _ANCHOR_W * jnp.exp(x[c0 + 3:c0 + 4, :])
        dec_ref[a, 4:5, :] = _ANCHOR_H * jnp.exp(x[c0 + 4:c0 + 5, :])
        dec_ref[a, 5:6, :] = (a + jax.nn.sigmoid(x[c0 + 5:c0 + 6, :])) * _THETA_MARGIN
        masks.append((conf >= thr).astype(jnp.int32).reshape(_NCHUNK, _CHW))

    m0, m1, m2 = masks
    pa1 = m0 + m1
    t = pa1 + m2
    pexc = _cumsum_lanes(t) - t
    d0 = jnp.where(m0 == 1, pexc + m0 - 1, -1)
    d1 = jnp.where(m1 == 1, pexc + pa1 - 1, -1)
    d2 = jnp.where(m2 == 1, pexc + t - 1, -1)

    iota_d = jax.lax.broadcasted_iota(jnp.int32, (_CHUNK_ROWS, _CHUNK_ROWS), 0)

    def dma(c, cnt, goff):
        return pltpu.make_async_copy(
            stages.at[c % _DEPTH, pl.ds(0, cnt), :],
            out_ref.at[pl.ds(goff, cnt), :],
            sems.at[c % _DEPTH],
        )

    prev = []
    for c in range(_NCHUNK):
        dest = jnp.concatenate(
            [d0[c:c + 1, :], d1[c:c + 1, :], d2[c:c + 1, :]], axis=1)
        sel = (iota_d == dest).astype(jnp.float32)
        tile6 = jnp.concatenate(
            [dec_ref[a, 0:6, c * _CHW:(c + 1) * _CHW] for a in range(_NA)],
            axis=1)
        outc = jax.lax.dot_general(
            sel, tile6, (((1,), (1,)), ((), ())),
            preferred_element_type=jnp.float32,
            precision=jax.lax.Precision.HIGHEST)
        if len(prev) >= _DEPTH:
            pc, pcnt, pgoff = prev.pop(0)

            @pl.when(pcnt > 0)
            def _():
                dma(pc, pcnt, pgoff).wait()
        stages[c % _DEPTH] = outc
        i = step * _NCHUNK + c
        goff = offs_ref[i]
        cnt = offs_ref[i + 1] - goff

        @pl.when(cnt > 0)
        def _():
            dma(c, cnt, goff).start()
        prev.append((c, cnt, goff))

    for pc, pcnt, pgoff in prev:
        @pl.when(pcnt > 0)
        def _():
            dma(pc, pcnt, pgoff).wait()


def kernel(output, confidence_threshold):
    N, C, H, W = output.shape
    x = output.reshape(N, C, _HW)
    thr = confidence_threshold.reshape(1)
    nrows = N * _HW * _NA

    counts = pl.pallas_call(
        _count_body,
        grid=(N,),
        in_specs=[
            pl.BlockSpec((1, C, _HW), lambda n: (n, 0, 0)),
            pl.BlockSpec(memory_space=pltpu.SMEM),
        ],
        out_specs=pl.BlockSpec((1, _HW // _CHW, 1), lambda n: (n, 0, 0)),
        out_shape=jax.ShapeDtypeStruct((N, _HW // _CHW, 1), jnp.int32),
        compiler_params=pltpu.CompilerParams(
            dimension_semantics=("parallel",),
        ),
    )(x, thr)

    offs = jnp.concatenate(
        [jnp.zeros((1,), jnp.int32),
         jnp.cumsum(counts.reshape(-1), dtype=jnp.int32)])

    outz = jnp.zeros((nrows, 6), jnp.float32)

    compacted = pl.pallas_call(
        _scatter_body,
        grid_spec=pltpu.PrefetchScalarGridSpec(
            num_scalar_prefetch=1,
            grid=(N, _NB),
            in_specs=[
                pl.BlockSpec((1, C, _HWBLK), lambda n, b, offs: (n, 0, b)),
                pl.BlockSpec(memory_space=pltpu.SMEM),
                pl.BlockSpec(memory_space=pl.ANY),
            ],
            out_specs=pl.BlockSpec(memory_space=pl.ANY),
            scratch_shapes=[
                pltpu.VMEM((_NA, 8, _HWBLK), jnp.float32),
                pltpu.VMEM((_DEPTH, _CHUNK_ROWS, 6), jnp.float32),
                pltpu.SemaphoreType.DMA((_DEPTH,)),
            ],
        ),
        out_shape=jax.ShapeDtypeStruct((nrows, 6), jnp.float32),
        input_output_aliases={3: 0},
        compiler_params=pltpu.CompilerParams(
            dimension_semantics=("parallel", "arbitrary"),
        ),
    )(offs, x, thr, outz)
    return compacted
